# baseline (speedup 1.0000x reference)
import functools

import jax
import jax.numpy as jnp
from jax import lax
from jax.experimental import pallas as pl
from jax.experimental.pallas import tpu as pltpu
from jax.experimental.pallas import tpu_sc as plsc

E = 16
K = 8


def _gating_kernel(task_ref, gw_ref, gb_ref,
                   probs_ref, mask_ref, sel_idx_ref, sel_gate_ref):
    t = task_ref[...]
    h = t * jax.nn.sigmoid(t)
    logits = jnp.dot(h, gw_ref[...], preferred_element_type=jnp.float32)
    logits = logits + gb_ref[...]
    m = jnp.max(logits, axis=1, keepdims=True)
    ex = jnp.exp(logits - m)
    p = ex / jnp.sum(ex, axis=1, keepdims=True)
    probs_ref[...] = p

    col = jax.lax.broadcasted_iota(jnp.int32, (E, E), 1)
    rank = jnp.zeros((E, E), jnp.int32)
    for j in range(E):
        pj = p[:, j:j + 1]
        gt = (pj > p).astype(jnp.int32)
        eq = jnp.logical_and(pj == p, col > j).astype(jnp.int32)
        rank = rank + gt + eq
    mask = (rank < K).astype(jnp.float32)
    mask_ref[...] = mask

    m0 = mask[0:1, :]
    c0 = p[0:1, :] * m0
    row = jax.lax.broadcasted_iota(jnp.int32, (E, E), 0)
    tri = (row <= col).astype(jnp.float32)
    pos = jnp.dot(m0, tri, preferred_element_type=jnp.float32) - 1.0
    kk = jax.lax.broadcasted_iota(jnp.int32, (K, E), 0).astype(jnp.float32)
    pos_b = jnp.broadcast_to(pos, (K, E))
    onehot = jnp.where(
        jnp.logical_and(pos_b == kk, jnp.broadcast_to(m0, (K, E)) > 0.5),
        1.0, 0.0)
    cols_f = jax.lax.broadcasted_iota(jnp.int32, (K, E), 1).astype(jnp.float32)
    sel_idx_ref[...] = jnp.sum(onehot * cols_f, axis=1,
                               keepdims=True).astype(jnp.int32)
    sel_gate_ref[...] = jnp.sum(onehot * jnp.broadcast_to(c0, (K, E)),
                                axis=1, keepdims=True)


def _moe_kernel(sel_ref, gate_ref, x_ref, *refs):
    w_refs = refs[:K]
    o_ref = refs[K]
    wc_ref = refs[K + 1]
    i = pl.program_id(0)

    @pl.when(i < 2)
    def _():
        acc = gate_ref[0] * w_refs[0][0]
        for j in range(1, K):
            acc += gate_ref[j] * w_refs[j][0]
        wc_ref[pl.ds(i * 512, 512), :] = acc.astype(jnp.bfloat16)

    @pl.when(i >= 2)
    def _():
        xb = x_ref[...].astype(jnp.bfloat16)
        o_ref[...] = 1.0 + jnp.dot(xb, wc_ref[...],
                                   preferred_element_type=jnp.float32)


def _make_sc_ones(total, row0):
    info = plsc.get_sparse_core_info()
    nw = info.num_cores * info.num_subcores
    per_tile = (total - row0) // nw
    ch = 16384
    n_dma = per_tile // ch
    mesh = plsc.VectorSubcoreMesh(core_axis_name="c", subcore_axis_name="s")

    @functools.partial(
        pl.kernel, mesh=mesh,
        out_type=jax.ShapeDtypeStruct((total,), jnp.float32),
        scratch_types=[pltpu.VMEM((ch,), jnp.float32)],
    )
    def sc_ones(out_hbm, buf):
        wid = lax.axis_index("s") * info.num_cores + lax.axis_index("c")

        def fill(i, carry):
            buf[pl.ds(i * 16, 16)] = jnp.full((16,), 1.0, jnp.float32)
            return carry

        lax.fori_loop(0, ch // 16, fill, 0)
        base = row0 + wid * per_tile
        for j in range(n_dma):
            pltpu.sync_copy(buf, out_hbm.at[pl.ds(base + j * ch, ch)])

    return sc_ones


def kernel(x, task_full, gate_w, gate_b, expert_w):
    B, L, D = x.shape

    ones_flat = _make_sc_ones(B * L * D, L * D)()
    ones_nd = ones_flat.reshape(B, L, D)

    probs, mask, sel_idx, sel_gate = pl.pallas_call(
        _gating_kernel,
        out_shape=(
            jax.ShapeDtypeStruct((E, E), jnp.float32),
            jax.ShapeDtypeStruct((E, E), jnp.float32),
            jax.ShapeDtypeStruct((K, 1), jnp.int32),
            jax.ShapeDtypeStruct((K, 1), jnp.float32),
        ),
    )(task_full, gate_w, gate_b.reshape(1, E))

    sel_idx = sel_idx.reshape(K)
    sel_gate = sel_gate.reshape(K)

    BM = 1024

    def x_idx(i, sel):
        return (jnp.clip(i - 2, 0, 1), 0)

    def w_idx_maker(j):
        def w_idx(i, sel):
            return (sel[j], jnp.minimum(i, 1), 0)
        return w_idx

    def out_idx(i, sel):
        return (jnp.clip(i - 2, 0, 1), 0)

    w_specs = [pl.BlockSpec((1, 512, D), w_idx_maker(j)) for j in range(K)]

    y0 = pl.pallas_call(
        _moe_kernel,
        grid_spec=pltpu.PrefetchScalarGridSpec(
            num_scalar_prefetch=1,
            grid=(4,),
            in_specs=[
                pl.BlockSpec(memory_space=pltpu.SMEM),
                pl.BlockSpec((BM, D), x_idx),
            ] + w_specs,
            out_specs=pl.BlockSpec((BM, D), out_idx),
            scratch_shapes=[
                pltpu.VMEM((D, D), jnp.bfloat16),
            ],
        ),
        out_shape=jax.ShapeDtypeStruct((L, D), jnp.float32),
        compiler_params=pltpu.CompilerParams(
            dimension_semantics=("arbitrary",)),
    )(sel_idx, sel_gate, x[0], *([expert_w] * K))

    out = lax.dynamic_update_slice(ones_nd, y0[None], (0, 0, 0))
    return out, probs[0], mask


# speedup vs baseline: 2.0354x; 2.0354x over previous
import jax
import jax.numpy as jnp
from jax.experimental import pallas as pl
from jax.experimental.pallas import tpu as pltpu

E = 16
K = 8


def _gating_kernel(task_ref, gw_ref, gb_ref,
                   probs_ref, mask_ref, sel_idx_ref, sel_gate_ref):
    t = task_ref[...]
    h = t * jax.nn.sigmoid(t)
    logits = jnp.dot(h, gw_ref[...], preferred_element_type=jnp.float32)
    logits = logits + gb_ref[...]
    m = jnp.max(logits, axis=1, keepdims=True)
    ex = jnp.exp(logits - m)
    p = ex / jnp.sum(ex, axis=1, keepdims=True)
    probs_ref[...] = p

    col = jax.lax.broadcasted_iota(jnp.int32, (E, E), 1)
    rank = jnp.zeros((E, E), jnp.int32)
    for j in range(E):
        pj = p[:, j:j + 1]
        gt = (pj > p).astype(jnp.int32)
        eq = jnp.logical_and(pj == p, col > j).astype(jnp.int32)
        rank = rank + gt + eq
    mask = (rank < K).astype(jnp.float32)
    mask_ref[...] = mask

    m0 = mask[0:1, :]
    c0 = p[0:1, :] * m0
    row = jax.lax.broadcasted_iota(jnp.int32, (E, E), 0)
    tri = (row <= col).astype(jnp.float32)
    pos = jnp.dot(m0, tri, preferred_element_type=jnp.float32) - 1.0
    kk = jax.lax.broadcasted_iota(jnp.int32, (K, E), 0).astype(jnp.float32)
    pos_b = jnp.broadcast_to(pos, (K, E))
    onehot = jnp.where(
        jnp.logical_and(pos_b == kk, jnp.broadcast_to(m0, (K, E)) > 0.5),
        1.0, 0.0)
    cols_f = jax.lax.broadcasted_iota(jnp.int32, (K, E), 1).astype(jnp.float32)
    sel_idx_ref[...] = jnp.sum(onehot * cols_f, axis=1,
                               keepdims=True).astype(jnp.int32)
    sel_gate_ref[...] = jnp.sum(onehot * jnp.broadcast_to(c0, (K, E)),
                                axis=1, keepdims=True)


def _moe_kernel(sel_ref, gate_ref, x_ref, *refs):
    w_refs = refs[:K]
    o_ref = refs[K]
    wc_ref = refs[K + 1]
    i = pl.program_id(0)

    @pl.when(i < 2)
    def _():
        acc = gate_ref[0] * w_refs[0][0]
        for j in range(1, K):
            acc += gate_ref[j] * w_refs[j][0]
        wc_ref[pl.ds(i * 512, 512), :] = acc.astype(jnp.bfloat16)
        o_ref[...] = jnp.ones_like(o_ref)

    @pl.when(jnp.logical_and(i >= 2, i < 4))
    def _():
        xb = x_ref[...].astype(jnp.bfloat16)
        o_ref[...] = (1.0 + jnp.dot(xb, wc_ref[...],
                                    preferred_element_type=jnp.float32))[None]

    @pl.when(i >= 4)
    def _():
        o_ref[...] = jnp.ones_like(o_ref)


def kernel(x, task_full, gate_w, gate_b, expert_w):
    B, L, D = x.shape

    probs, mask, sel_idx, sel_gate = pl.pallas_call(
        _gating_kernel,
        out_shape=(
            jax.ShapeDtypeStruct((E, E), jnp.float32),
            jax.ShapeDtypeStruct((E, E), jnp.float32),
            jax.ShapeDtypeStruct((K, 1), jnp.int32),
            jax.ShapeDtypeStruct((K, 1), jnp.float32),
        ),
    )(task_full, gate_w, gate_b.reshape(1, E))

    sel_idx = sel_idx.reshape(K)
    sel_gate = sel_gate.reshape(K)

    BM = 1024
    n_steps = 8

    def x_idx(i, sel):
        return (jnp.clip(i - 2, 0, 1), 0)

    def w_idx_maker(j):
        def w_idx(i, sel):
            return (sel[j], jnp.minimum(i, 1), 0)
        return w_idx

    def out_idx(i, sel):
        b = jnp.where(i < 2, 1, jnp.where(i < 4, 0, i // 2))
        return (b, i % 2, 0)

    w_specs = [pl.BlockSpec((1, 512, D), w_idx_maker(j)) for j in range(K)]

    out = pl.pallas_call(
        _moe_kernel,
        grid_spec=pltpu.PrefetchScalarGridSpec(
            num_scalar_prefetch=1,
            grid=(n_steps,),
            in_specs=[
                pl.BlockSpec(memory_space=pltpu.SMEM),
                pl.BlockSpec((BM, D), x_idx),
            ] + w_specs,
            out_specs=pl.BlockSpec((1, BM, D), out_idx),
            scratch_shapes=[
                pltpu.VMEM((D, D), jnp.bfloat16),
            ],
        ),
        out_shape=jax.ShapeDtypeStruct((B, L, D), jnp.float32),
        compiler_params=pltpu.CompilerParams(
            dimension_semantics=("arbitrary",)),
    )(sel_idx, sel_gate, x[0], *([expert_w] * K))

    return out, probs[0], mask
